# baseline (speedup 1.0000x reference)
import functools

import jax
import jax.numpy as jnp
from jax.experimental import pallas as pl
from jax.experimental.pallas import tpu as pltpu


def _tgc_kernel(adj_ref, x_ref, M_ref, W_ref, out_ref, gall_ref, q_ref,
                *, T, N, F_IN, F_OUT, BW):
    i = pl.program_id(0)
    s = pl.program_id(1)

    @pl.when(jnp.logical_and(i == 0, s == 0))
    def _init_gall():
        for t in range(T):
            xt = M_ref[t, 0] * x_ref[0]
            for tau in range(1, T):
                xt = xt + M_ref[t, tau] * x_ref[tau]
            g = jax.lax.dot(xt, W_ref[...],
                            precision=jax.lax.Precision.HIGHEST,
                            preferred_element_type=jnp.float32)
            gall_ref[:, t * F_OUT:(t + 1) * F_OUT] = g.astype(jnp.bfloat16)

    a = adj_ref[0].astype(jnp.bfloat16)
    p = jax.lax.dot(a, gall_ref[...],
                    preferred_element_type=jnp.float32)

    gid = jax.lax.broadcasted_iota(jnp.int32, (1, T * F_OUT), 1) // F_OUT
    cvec = jnp.zeros((1, T * F_OUT), jnp.float32)
    for j in range(BW):
        t = s + j
        w = jnp.where(t < T, M_ref[jnp.minimum(t, T - 1), s], 0.0)
        cvec = cvec + jnp.where(gid == t, w, 0.0)
    contrib = p * cvec

    @pl.when(s == 0)
    def _():
        q_ref[...] = contrib

    @pl.when(s > 0)
    def _():
        q_ref[...] = q_ref[...] + contrib

    @pl.when(s == T - 1)
    def _finalize():
        q = q_ref[...]
        for t in range(T):
            out_ref[t] = q[:, t * F_OUT:(t + 1) * F_OUT]


@jax.jit
def kernel(adj, x, M, W):
    T, N, _ = adj.shape
    F_IN = x.shape[2]
    F_OUT = W.shape[1]
    BW = 3
    bN = 512
    body = functools.partial(_tgc_kernel, T=T, N=N, F_IN=F_IN, F_OUT=F_OUT,
                             BW=BW)
    return pl.pallas_call(
        body,
        grid=(N // bN, T),
        in_specs=[
            pl.BlockSpec((1, bN, N), lambda i, s: (s, i, 0)),
            pl.BlockSpec((T, N, F_IN), lambda i, s: (0, 0, 0)),
            pl.BlockSpec(memory_space=pltpu.SMEM),
            pl.BlockSpec((F_IN, F_OUT), lambda i, s: (0, 0)),
        ],
        out_specs=pl.BlockSpec((T, bN, F_OUT), lambda i, s: (0, i, 0)),
        out_shape=jax.ShapeDtypeStruct((T, N, F_OUT), jnp.float32),
        scratch_shapes=[
            pltpu.VMEM((N, T * F_OUT), jnp.bfloat16),
            pltpu.VMEM((bN, T * F_OUT), jnp.float32),
        ],
    )(adj, x, M, W)


# speedup vs baseline: 1.1262x; 1.1262x over previous
import functools

import jax
import jax.numpy as jnp
from jax.experimental import pallas as pl
from jax.experimental.pallas import tpu as pltpu

_NBUF = 12
_BN = 256


def _issue(adj_hbm, buf_ref, sem, b, T, bN):
    i_b = b // T
    s_b = b % T
    slot = b % _NBUF
    pltpu.make_async_copy(
        adj_hbm.at[s_b, pl.ds(i_b * bN, bN), :],
        buf_ref.at[slot],
        sem.at[slot],
    ).start()


def _tgc_kernel(adj_hbm, x_ref, M_ref, W_ref, out_ref, buf_ref, gall_ref,
                q_ref, sem, *, T, N, F_IN, F_OUT, BW, bN):
    c = pl.program_id(0)
    total = (N // bN) * T
    s = c % T

    @pl.when(c == 0)
    def _prologue():
        for b in range(min(_NBUF, total)):
            _issue(adj_hbm, buf_ref, sem, b, T, bN)
        for t in range(T):
            xt = M_ref[t, 0] * x_ref[0]
            for tau in range(1, T):
                xt = xt + M_ref[t, tau] * x_ref[tau]
            g = jax.lax.dot(xt, W_ref[...],
                            precision=jax.lax.Precision.HIGHEST,
                            preferred_element_type=jnp.float32)
            gall_ref[:, t * F_OUT:(t + 1) * F_OUT] = g.astype(jnp.bfloat16)

    @pl.when(jnp.logical_and(c > 0, c + _NBUF - 1 < total))
    def _prefetch():
        _issue(adj_hbm, buf_ref, sem, c + _NBUF - 1, T, bN)

    slot = c % _NBUF
    i_c = c // T
    pltpu.make_async_copy(
        adj_hbm.at[s, pl.ds(i_c * bN, bN), :],
        buf_ref.at[slot],
        sem.at[slot],
    ).wait()

    a = buf_ref[slot].astype(jnp.bfloat16)
    p = jax.lax.dot(a, gall_ref[...],
                    preferred_element_type=jnp.float32)

    gid = jax.lax.broadcasted_iota(jnp.int32, (1, T * F_OUT), 1) // F_OUT
    cvec = jnp.zeros((1, T * F_OUT), jnp.float32)
    for j in range(BW):
        t = s + j
        w = jnp.where(t < T, M_ref[jnp.minimum(t, T - 1), s], 0.0)
        cvec = cvec + jnp.where(gid == t, w, 0.0)
    contrib = p * cvec

    @pl.when(s == 0)
    def _():
        q_ref[...] = contrib

    @pl.when(s > 0)
    def _():
        q_ref[...] = q_ref[...] + contrib

    @pl.when(s == T - 1)
    def _finalize():
        q = q_ref[...]
        for t in range(T):
            out_ref[t] = q[:, t * F_OUT:(t + 1) * F_OUT]


@jax.jit
def kernel(adj, x, M, W):
    T, N, _ = adj.shape
    F_IN = x.shape[2]
    F_OUT = W.shape[1]
    BW = 3
    bN = _BN
    body = functools.partial(_tgc_kernel, T=T, N=N, F_IN=F_IN, F_OUT=F_OUT,
                             BW=BW, bN=bN)
    return pl.pallas_call(
        body,
        grid=((N // bN) * T,),
        in_specs=[
            pl.BlockSpec(memory_space=pltpu.MemorySpace.HBM),
            pl.BlockSpec((T, N, F_IN), lambda c: (0, 0, 0)),
            pl.BlockSpec(memory_space=pltpu.SMEM),
            pl.BlockSpec((F_IN, F_OUT), lambda c: (0, 0)),
        ],
        out_specs=pl.BlockSpec((T, bN, F_OUT), lambda c: (0, c // T, 0)),
        out_shape=jax.ShapeDtypeStruct((T, N, F_OUT), jnp.float32),
        scratch_shapes=[
            pltpu.VMEM((_NBUF, bN, N), jnp.float32),
            pltpu.VMEM((N, T * F_OUT), jnp.bfloat16),
            pltpu.VMEM((bN, T * F_OUT), jnp.float32),
            pltpu.SemaphoreType.DMA((_NBUF,)),
        ],
    )(adj, x, M, W)


# speedup vs baseline: 1.2281x; 1.0905x over previous
import functools

import jax
import jax.numpy as jnp
from jax.experimental import pallas as pl
from jax.experimental.pallas import tpu as pltpu

_NBUF = 12
_BN = 256


def _issue(adj_hbm, buf_ref, sem, b, T, bN):
    i_b = b // T
    s_b = b % T
    slot = b % _NBUF
    pltpu.make_async_copy(
        adj_hbm.at[s_b, pl.ds(i_b * bN, bN), :],
        buf_ref.at[slot],
        sem.at[slot],
    ).start()


def _tgc_kernel(adj_hbm, x_ref, ubig_ref, out_ref, buf_ref, gbig_ref,
                xcat_ref, q_ref, sem, *, T, N, F_IN, F_OUT, bN):
    c = pl.program_id(0)
    total = (N // bN) * T
    s = c % T

    @pl.when(c == 0)
    def _prologue():
        for b in range(min(_NBUF, total)):
            _issue(adj_hbm, buf_ref, sem, b, T, bN)
        for tau in range(T):
            xcat_ref[:, tau * F_IN:(tau + 1) * F_IN] = x_ref[tau]
        xc = xcat_ref[...]
        for sb in range(T):
            gbig_ref[sb] = jax.lax.dot(xc, ubig_ref[sb],
                                       preferred_element_type=jnp.float32)

    @pl.when(jnp.logical_and(c > 0, c + _NBUF - 1 < total))
    def _prefetch():
        _issue(adj_hbm, buf_ref, sem, c + _NBUF - 1, T, bN)

    slot = c % _NBUF
    i_c = c // T
    pltpu.make_async_copy(
        adj_hbm.at[s, pl.ds(i_c * bN, bN), :],
        buf_ref.at[slot],
        sem.at[slot],
    ).wait()

    p = jax.lax.dot(buf_ref[slot], gbig_ref[s],
                    preferred_element_type=jnp.float32)

    @pl.when(s == 0)
    def _():
        q_ref[...] = p

    @pl.when(s > 0)
    def _():
        q_ref[...] = q_ref[...] + p

    @pl.when(s == T - 1)
    def _finalize():
        q = q_ref[...]
        for t in range(T):
            out_ref[t] = q[:, t * F_OUT:(t + 1) * F_OUT]


@jax.jit
def kernel(adj, x, M, W):
    T, N, _ = adj.shape
    F_IN = x.shape[2]
    F_OUT = W.shape[1]
    BW = 3
    bN = _BN
    rows = jnp.arange(T)[:, None]
    cols = jnp.arange(T)[None, :]
    band = (cols <= rows) & (cols >= rows - BW + 1)
    Mb = jnp.where(band, M, jnp.zeros_like(M))
    U = jnp.kron(M.T, W)
    cvec = jnp.repeat(Mb.T, F_OUT, axis=1)
    ubig = U[None, :, :] * cvec[:, None, :]

    body = functools.partial(_tgc_kernel, T=T, N=N, F_IN=F_IN, F_OUT=F_OUT,
                             bN=bN)
    return pl.pallas_call(
        body,
        grid=((N // bN) * T,),
        in_specs=[
            pl.BlockSpec(memory_space=pltpu.MemorySpace.HBM),
            pl.BlockSpec((T, N, F_IN), lambda c: (0, 0, 0)),
            pl.BlockSpec((T, T * F_IN, T * F_OUT), lambda c: (0, 0, 0)),
        ],
        out_specs=pl.BlockSpec((T, bN, F_OUT), lambda c: (0, c // T, 0)),
        out_shape=jax.ShapeDtypeStruct((T, N, F_OUT), jnp.float32),
        scratch_shapes=[
            pltpu.VMEM((_NBUF, bN, N), jnp.float32),
            pltpu.VMEM((T, N, T * F_OUT), jnp.float32),
            pltpu.VMEM((N, T * F_IN), jnp.float32),
            pltpu.VMEM((bN, T * F_OUT), jnp.float32),
            pltpu.SemaphoreType.DMA((_NBUF,)),
        ],
    )(adj, x, ubig)


# speedup vs baseline: 1.2309x; 1.0023x over previous
import functools

import jax
import jax.numpy as jnp
from jax.experimental import pallas as pl
from jax.experimental.pallas import tpu as pltpu

_NBUF = 14
_BN = 256


def _issue(adj_hbm, buf_ref, sem, b, T, bN):
    i_b = b // T
    s_b = b % T
    slot = b % _NBUF
    pltpu.make_async_copy(
        adj_hbm.at[s_b, pl.ds(i_b * bN, bN), :],
        buf_ref.at[slot],
        sem.at[slot],
    ).start()


def _tgc_kernel(adj_hbm, x_ref, ubig_ref, out_ref, buf_ref, gbig_ref,
                xcat_ref, q_ref, sem, *, T, N, F_IN, F_OUT, bN):
    c = pl.program_id(0)
    total = (N // bN) * T
    s = c % T

    @pl.when(c == 0)
    def _prologue():
        for b in range(min(_NBUF, total)):
            _issue(adj_hbm, buf_ref, sem, b, T, bN)
        for tau in range(T):
            xcat_ref[:, tau * F_IN:(tau + 1) * F_IN] = x_ref[tau]
        xc = xcat_ref[...]
        for sb in range(T):
            gbig_ref[sb] = jax.lax.dot(xc, ubig_ref[sb],
                                       preferred_element_type=jnp.float32)

    @pl.when(jnp.logical_and(c > 0, c + _NBUF - 1 < total))
    def _prefetch():
        _issue(adj_hbm, buf_ref, sem, c + _NBUF - 1, T, bN)

    slot = c % _NBUF
    i_c = c // T
    pltpu.make_async_copy(
        adj_hbm.at[s, pl.ds(i_c * bN, bN), :],
        buf_ref.at[slot],
        sem.at[slot],
    ).wait()

    p = jax.lax.dot(buf_ref[slot], gbig_ref[s],
                    preferred_element_type=jnp.float32)

    @pl.when(s == 0)
    def _():
        q_ref[...] = p

    @pl.when(s > 0)
    def _():
        q_ref[...] = q_ref[...] + p

    @pl.when(s == T - 1)
    def _finalize():
        q = q_ref[...]
        for t in range(T):
            out_ref[t] = q[:, t * F_OUT:(t + 1) * F_OUT]


@jax.jit
def kernel(adj, x, M, W):
    T, N, _ = adj.shape
    F_IN = x.shape[2]
    F_OUT = W.shape[1]
    BW = 3
    bN = _BN
    rows = jnp.arange(T)[:, None]
    cols = jnp.arange(T)[None, :]
    band = (cols <= rows) & (cols >= rows - BW + 1)
    Mb = jnp.where(band, M, jnp.zeros_like(M))
    U = jnp.kron(M.T, W)
    cvec = jnp.repeat(Mb.T, F_OUT, axis=1)
    ubig = U[None, :, :] * cvec[:, None, :]

    body = functools.partial(_tgc_kernel, T=T, N=N, F_IN=F_IN, F_OUT=F_OUT,
                             bN=bN)
    return pl.pallas_call(
        body,
        grid=((N // bN) * T,),
        in_specs=[
            pl.BlockSpec(memory_space=pltpu.MemorySpace.HBM),
            pl.BlockSpec((T, N, F_IN), lambda c: (0, 0, 0)),
            pl.BlockSpec((T, T * F_IN, T * F_OUT), lambda c: (0, 0, 0)),
        ],
        out_specs=pl.BlockSpec((T, bN, F_OUT), lambda c: (0, c // T, 0)),
        out_shape=jax.ShapeDtypeStruct((T, N, F_OUT), jnp.float32),
        scratch_shapes=[
            pltpu.VMEM((_NBUF, bN, N), jnp.float32),
            pltpu.VMEM((T, N, T * F_OUT), jnp.float32),
            pltpu.VMEM((N, T * F_IN), jnp.float32),
            pltpu.VMEM((bN, T * F_OUT), jnp.float32),
            pltpu.SemaphoreType.DMA((_NBUF,)),
        ],
    )(adj, x, ubig)


# speedup vs baseline: 1.2657x; 1.0283x over previous
import functools

import jax
import jax.numpy as jnp
from jax.experimental import pallas as pl
from jax.experimental.pallas import tpu as pltpu

_NBUF = 6
_BN = 512


def _issue(adj_hbm, buf_ref, sem, b, T, bN):
    i_b = b // T
    s_b = b % T
    slot = b % _NBUF
    pltpu.make_async_copy(
        adj_hbm.at[s_b, pl.ds(i_b * bN, bN), :],
        buf_ref.at[slot],
        sem.at[slot],
    ).start()


def _tgc_kernel(adj_hbm, x_ref, ubig_ref, out_ref, buf_ref, gbig_ref,
                xcat_ref, q_ref, sem, *, T, N, F_IN, F_OUT, bN):
    c = pl.program_id(0)
    total = (N // bN) * T
    s = c % T

    @pl.when(c == 0)
    def _prologue():
        for b in range(min(_NBUF, total)):
            _issue(adj_hbm, buf_ref, sem, b, T, bN)
        for tau in range(T):
            xcat_ref[:, tau * F_IN:(tau + 1) * F_IN] = x_ref[tau]
        xc = xcat_ref[...]
        for sb in range(T):
            gbig_ref[sb] = jax.lax.dot(xc, ubig_ref[sb],
                                       preferred_element_type=jnp.float32)

    @pl.when(jnp.logical_and(c > 0, c + _NBUF - 1 < total))
    def _prefetch():
        _issue(adj_hbm, buf_ref, sem, c + _NBUF - 1, T, bN)

    slot = c % _NBUF
    i_c = c // T
    pltpu.make_async_copy(
        adj_hbm.at[s, pl.ds(i_c * bN, bN), :],
        buf_ref.at[slot],
        sem.at[slot],
    ).wait()

    p = jax.lax.dot(buf_ref[slot], gbig_ref[s],
                    preferred_element_type=jnp.float32)

    @pl.when(s == 0)
    def _():
        q_ref[...] = p

    @pl.when(s > 0)
    def _():
        q_ref[...] = q_ref[...] + p

    @pl.when(s == T - 1)
    def _finalize():
        q = q_ref[...]
        for t in range(T):
            out_ref[t] = q[:, t * F_OUT:(t + 1) * F_OUT]


@jax.jit
def kernel(adj, x, M, W):
    T, N, _ = adj.shape
    F_IN = x.shape[2]
    F_OUT = W.shape[1]
    BW = 3
    bN = _BN
    rows = jnp.arange(T)[:, None]
    cols = jnp.arange(T)[None, :]
    band = (cols <= rows) & (cols >= rows - BW + 1)
    Mb = jnp.where(band, M, jnp.zeros_like(M))
    U = jnp.kron(M.T, W)
    cvec = jnp.repeat(Mb.T, F_OUT, axis=1)
    ubig = U[None, :, :] * cvec[:, None, :]

    body = functools.partial(_tgc_kernel, T=T, N=N, F_IN=F_IN, F_OUT=F_OUT,
                             bN=bN)
    return pl.pallas_call(
        body,
        grid=((N // bN) * T,),
        in_specs=[
            pl.BlockSpec(memory_space=pltpu.MemorySpace.HBM),
            pl.BlockSpec((T, N, F_IN), lambda c: (0, 0, 0)),
            pl.BlockSpec((T, T * F_IN, T * F_OUT), lambda c: (0, 0, 0)),
        ],
        out_specs=pl.BlockSpec((T, bN, F_OUT), lambda c: (0, c // T, 0)),
        out_shape=jax.ShapeDtypeStruct((T, N, F_OUT), jnp.float32),
        scratch_shapes=[
            pltpu.VMEM((_NBUF, bN, N), jnp.float32),
            pltpu.VMEM((T, N, T * F_OUT), jnp.float32),
            pltpu.VMEM((N, T * F_IN), jnp.float32),
            pltpu.VMEM((bN, T * F_OUT), jnp.float32),
            pltpu.SemaphoreType.DMA((_NBUF,)),
        ],
    )(adj, x, ubig)
